# baseline (speedup 1.0000x reference)
import functools
import jax
import jax.numpy as jnp
from jax import lax
from jax.experimental import pallas as pl
from jax.experimental.pallas import tpu as pltpu
from jax.experimental.pallas import tpu_sc as plsc

D = 64
S = 200
SP = 208
CH0 = 104
CH1 = S - CH0
NG = SP // 16


@jax.jit
def _run(sl, embed, W, bias):
    B = embed.shape[0]
    info = plsc.get_sparse_core_info()
    NC, NS = info.num_cores, info.num_subcores
    NW = NC * NS
    spw = B // NW
    mesh = plsc.VectorSubcoreMesh(core_axis_name="c", subcore_axis_name="s")

    @functools.partial(
        pl.kernel,
        out_type=jax.ShapeDtypeStruct((B, S), jnp.float32),
        mesh=mesh,
        compiler_params=pltpu.CompilerParams(
            needs_layout_passes=False, use_tc_tiling_on_sc=False),
        scratch_types=[
            pltpu.VMEM((spw, S), jnp.int32),
            pltpu.VMEM((spw, D), jnp.float32),
            pltpu.VMEM((2, SP, D), jnp.float32),
            pltpu.VMEM((2, SP, 1), jnp.float32),
            pltpu.VMEM((2, SP), jnp.float32),
            pltpu.SemaphoreType.DMA,
            pltpu.SemaphoreType.DMA,
            pltpu.SemaphoreType.DMA,
            pltpu.SemaphoreType.DMA,
            pltpu.SemaphoreType.DMA,
            pltpu.SemaphoreType.DMA,
        ],
    )
    def k(sl_hbm, embed_hbm, w_hbm, bias_hbm, out_hbm,
          idx_all, emb_all, rows_v, bias_v, out_stage,
          sw0, sw1, sb0, sb1, so0, so1):
        wid = lax.axis_index("s") * NC + lax.axis_index("c")
        base = wid * spw
        svecs = [lax.iota(jnp.int32, 16) + 16 * g for g in range(NG)]
        zvec = jnp.zeros((16,), jnp.int32)
        sems = ((sw0, sb0, so0), (sw1, sb1, so1))

        pltpu.sync_copy(sl_hbm.at[pl.ds(base, spw)], idx_all)
        pltpu.sync_copy(embed_hbm.at[pl.ds(base, spw)], emb_all)

        def mk_gathers(i, buf):
            sw, sb, _ = sems[buf]
            rb = rows_v.at[buf]
            bb = bias_v.at[buf]
            c0 = idx_all.at[i, pl.ds(0, CH0)]
            c1 = idx_all.at[i, pl.ds(CH0, CH1)]
            return (
                pltpu.make_async_copy(w_hbm.at[c0],
                                      rb.at[pl.ds(0, CH0)], sw),
                pltpu.make_async_copy(w_hbm.at[c1],
                                      rb.at[pl.ds(CH0, CH1)], sw),
                pltpu.make_async_copy(bias_hbm.at[c0],
                                      bb.at[pl.ds(0, CH0)], sb),
                pltpu.make_async_copy(bias_hbm.at[c1],
                                      bb.at[pl.ds(CH0, CH1)], sb),
            )

        def mk_out(i, buf):
            return pltpu.make_async_copy(
                out_stage.at[buf, pl.ds(0, S)], out_hbm.at[base + i],
                sems[buf][2])

        def issue(i, buf):
            for c in mk_gathers(i, buf):
                c.start()

        def drain(i, buf):
            for c in mk_gathers(i, buf):
                c.wait()

        def compute(i, buf):
            rb = rows_v.at[buf]
            bb = bias_v.at[buf]
            accs0 = tuple(
                plsc.load_gather(bb, [svecs[g], zvec]) for g in range(NG))
            isplat = zvec + i

            def dbody(d, accs):
                dsplat = zvec + d
                e = plsc.load_gather(emb_all, [isplat, dsplat])
                return tuple(
                    a + plsc.load_gather(rb, [svecs[g], dsplat]) * e
                    for g, a in enumerate(accs)
                )

            accs = lax.fori_loop(0, D, dbody, accs0)
            for g in range(NG):
                out_stage[buf, pl.ds(16 * g, 16)] = accs[g]

        issue(0, 0)

        def pair_body(j, carry):
            e, o, n = 2 * j, 2 * j + 1, 2 * j + 2
            issue(o, 1)
            drain(e, 0)

            @pl.when(j > 0)
            def _():
                mk_out(e - 2, 0).wait()

            compute(e, 0)
            mk_out(e, 0).start()

            @pl.when(n < spw)
            def _():
                issue(n, 0)

            drain(o, 1)

            @pl.when(j > 0)
            def _():
                mk_out(o - 2, 1).wait()

            compute(o, 1)
            mk_out(o, 1).start()
            return carry

        lax.fori_loop(0, spw // 2, pair_body, 0)
        mk_out(spw - 2, 0).wait()
        mk_out(spw - 1, 1).wait()

    return k(sl, embed, W, bias)


def kernel(embed, shortlist, W, b):
    return _run(shortlist, embed, W, b)


# speedup vs baseline: 1.5142x; 1.5142x over previous
import functools
import jax
import jax.numpy as jnp
from jax import lax
from jax.experimental import pallas as pl
from jax.experimental.pallas import tpu as pltpu
from jax.experimental.pallas import tpu_sc as plsc

D = 64
S = 200
SP = 208
CH0 = 128
CH1 = S - CH0
NG = SP // 16
PAD_ROW = 1000000


@jax.jit
def _run(sl2, embed, W, bias):
    B = embed.shape[0]
    info = plsc.get_sparse_core_info()
    NC, NS = info.num_cores, info.num_subcores
    NW = NC * NS
    spw = B // NW
    mesh = plsc.VectorSubcoreMesh(core_axis_name="c", subcore_axis_name="s")

    @functools.partial(
        pl.kernel,
        out_type=jax.ShapeDtypeStruct((2 * B, 128), jnp.float32),
        mesh=mesh,
        compiler_params=pltpu.CompilerParams(
            needs_layout_passes=False, use_tc_tiling_on_sc=False),
        scratch_types=[
            pltpu.VMEM((2 * spw, 128), jnp.int32),
            pltpu.VMEM((spw, D), jnp.float32),
            pltpu.VMEM((2, SP, D), jnp.float32),
            pltpu.VMEM((2, SP), jnp.float32),
            pltpu.VMEM((2, 256), jnp.float32),
            pltpu.SemaphoreType.DMA,
            pltpu.SemaphoreType.DMA,
            pltpu.SemaphoreType.DMA,
            pltpu.SemaphoreType.DMA,
            pltpu.SemaphoreType.DMA,
            pltpu.SemaphoreType.DMA,
        ],
    )
    def k(sl_hbm, embed_hbm, w_hbm, bias_hbm, out_hbm,
          idx_all, emb_all, rows_v, bias_v, out_stage,
          sw0, sw1, sb0, sb1, so0, so1):
        wid = lax.axis_index("s") * NC + lax.axis_index("c")
        base = wid * spw
        svecs = [lax.iota(jnp.int32, 16) + 16 * g for g in range(NG)]
        zvec = jnp.zeros((16,), jnp.int32)
        sems = ((sw0, sb0, so0), (sw1, sb1, so1))

        pltpu.sync_copy(sl_hbm.at[pl.ds(2 * base, 2 * spw)], idx_all)
        pltpu.sync_copy(embed_hbm.at[pl.ds(base, spw)], emb_all)

        def mk_gathers(i, buf):
            sw, sb, _ = sems[buf]
            rb = rows_v.at[buf]
            bb = bias_v.at[buf]
            c0 = idx_all.at[2 * i]
            c1 = idx_all.at[2 * i + 1, pl.ds(0, CH1)]
            return (
                pltpu.make_async_copy(w_hbm.at[c0],
                                      rb.at[pl.ds(0, CH0)], sw),
                pltpu.make_async_copy(w_hbm.at[c1],
                                      rb.at[pl.ds(CH0, CH1)], sw),
                pltpu.make_async_copy(bias_hbm.at[c0],
                                      bb.at[pl.ds(0, CH0)], sb),
                pltpu.make_async_copy(bias_hbm.at[c1],
                                      bb.at[pl.ds(CH0, CH1)], sb),
            )

        def mk_out(i, buf):
            half0 = pltpu.make_async_copy(
                out_stage.at[buf, pl.ds(0, 128)],
                out_hbm.at[2 * (base + i)], sems[buf][2])
            half1 = pltpu.make_async_copy(
                out_stage.at[buf, pl.ds(128, 128)],
                out_hbm.at[2 * (base + i) + 1], sems[buf][2])
            return (half0, half1)

        def issue(i, buf):
            for c in mk_gathers(i, buf):
                c.start()

        def drain(i, buf):
            for c in mk_gathers(i, buf):
                c.wait()

        def out_start(i, buf):
            for c in mk_out(i, buf):
                c.start()

        def out_drain(i, buf):
            for c in mk_out(i, buf):
                c.wait()

        def compute(i, buf):
            rb = rows_v.at[buf]
            accs0 = tuple(bias_v[buf, pl.ds(16 * g, 16)] for g in range(NG))
            isplat = zvec + i

            def dbody(d, accs):
                dsplat = zvec + d
                e = plsc.load_gather(emb_all, [isplat, dsplat])
                return tuple(
                    a + plsc.load_gather(rb, [svecs[g], dsplat]) * e
                    for g, a in enumerate(accs)
                )

            accs = lax.fori_loop(0, D, dbody, accs0)
            for g in range(NG):
                out_stage[buf, pl.ds(16 * g, 16)] = accs[g]

        issue(0, 0)

        def pair_body(j, carry):
            e, o, n = 2 * j, 2 * j + 1, 2 * j + 2
            issue(o, 1)
            drain(e, 0)

            @pl.when(j > 0)
            def _():
                out_drain(e - 2, 0)

            compute(e, 0)
            out_start(e, 0)

            @pl.when(n < spw)
            def _():
                issue(n, 0)

            drain(o, 1)

            @pl.when(j > 0)
            def _():
                out_drain(o - 2, 1)

            compute(o, 1)
            out_start(o, 1)
            return carry

        lax.fori_loop(0, spw // 2, pair_body, 0)
        out_drain(spw - 2, 0)
        out_drain(spw - 1, 1)

    return k(sl2, embed, W, bias)


def kernel(embed, shortlist, W, b):
    B = embed.shape[0]
    sl2 = jnp.pad(shortlist.astype(jnp.int32), ((0, 0), (0, 256 - S)),
                  constant_values=PAD_ROW).reshape(2 * B, 128)
    bias = b.reshape(-1)
    out2 = _run(sl2, embed, W, bias)
    return out2.reshape(B, 256)[:, :S]
